# baseline (speedup 1.0000x reference)
import jax
import jax.numpy as jnp
from jax import lax
from jax.experimental import pallas as pl
from jax.experimental.pallas import tpu as pltpu
from jax.experimental.pallas import tpu_sc as plsc

B, R, C = 2, 8192, 2048
N = B * R * C
K = int(0.9999 * N)
NW = 32
RW = (B * R) // NW
CR = 8
NCH = RW // CR
NB = 65536
L = 16


def _make_hist(pass1):
    def body(data_hbm, sel_hbm, out_hbm, hist_v, buf0, buf1, sel_v,
             sem0, sem1):
        cid = lax.axis_index("c")
        sid = lax.axis_index("s")
        wid = sid * 2 + cid
        b = wid // 16
        r0 = (wid % 16) * RW

        pltpu.sync_copy(sel_hbm, sel_v)
        sel = sel_v[...]

        @plsc.parallel_loop(0, NB // L, unroll=16)
        def _(i):
            hist_v[pl.ds(i * L, L)] = jnp.zeros((L,), jnp.int32)

        ones = jnp.ones((L,), jnp.int32)

        def process(buf):
            for j in range(CR):
                @plsc.parallel_loop(0, C // L, unroll=16)
                def _(i):
                    bits = plsc.bitcast(buf[j, pl.ds(i * L, L)], jnp.int32)
                    if pass1:
                        idx = lax.shift_right_logical(bits, 16)
                        plsc.addupdate_scatter(hist_v, [idx], ones)
                    else:
                        idx = bits & jnp.int32(0xFFFF)
                        msk = (bits & jnp.int32(0x7FFF0000)) == sel
                        plsc.addupdate_scatter(hist_v, [idx], ones, mask=msk)

        def start(buf, sem, chunk):
            pltpu.make_async_copy(
                data_hbm.at[b, pl.ds(r0 + chunk * CR, CR), :], buf,
                sem).start()

        def wait(buf, sem):
            pltpu.make_async_copy(
                data_hbm.at[b, pl.ds(r0, CR), :], buf, sem).wait()

        start(buf0, sem0, 0)

        def outer(g2, c):
            g = g2 * 2
            start(buf1, sem1, g + 1)
            wait(buf0, sem0)
            process(buf0)

            @pl.when(g + 2 < NCH)
            def _():
                start(buf0, sem0, g + 2)

            wait(buf1, sem1)
            process(buf1)
            return c

        lax.fori_loop(0, NCH // 2, outer, 0)

        pltpu.sync_copy(hist_v, out_hbm.at[wid])

    return pl.kernel(
        body,
        out_type=jax.ShapeDtypeStruct((NW, NB), jnp.int32),
        mesh=plsc.VectorSubcoreMesh(core_axis_name="c", subcore_axis_name="s"),
        compiler_params=pltpu.CompilerParams(needs_layout_passes=False),
        scratch_types=[
            pltpu.VMEM((NB,), jnp.int32),
            pltpu.VMEM((CR, C), jnp.float32),
            pltpu.VMEM((CR, C), jnp.float32),
            pltpu.VMEM((L,), jnp.int32),
            pltpu.SemaphoreType.DMA,
            pltpu.SemaphoreType.DMA,
        ],
    )


_hist1 = _make_hist(True)
_hist2 = _make_hist(False)


def _select_bin(hist, rank):
    c = jnp.cumsum(hist)
    bsel = jnp.argmax(c >= rank).astype(jnp.int32)
    below = jnp.where(bsel > 0, c[jnp.maximum(bsel - 1, 0)], 0)
    return bsel, below


def kernel(input):
    zero_sel = jnp.zeros((L,), jnp.int32)
    part1 = _hist1(input, zero_sel).sum(axis=0)
    h1 = part1[:NB // 2] + part1[NB // 2:]
    b1, below1 = _select_bin(h1, K)
    part2 = _hist2(input, zero_sel + (b1 << 16)).sum(axis=0)
    b2, _ = _select_bin(part2, K - below1)
    return lax.bitcast_convert_type((b1 << 16) | b2, jnp.float32)


# speedup vs baseline: 1.0301x; 1.0301x over previous
import jax
import jax.numpy as jnp
from jax import lax
from jax.experimental import pallas as pl
from jax.experimental.pallas import tpu as pltpu
from jax.experimental.pallas import tpu_sc as plsc

B, R, C = 2, 8192, 2048
N = B * R * C
K = int(0.9999 * N)
NW = 32
RW = (B * R) // NW
L = 16


def _make_hist(pass1):
    nb = 65536 if pass1 else 32768
    cr = 8 if pass1 else 16
    nch = RW // cr

    def body(data_hbm, sel_hbm, out_hbm, hist_v, buf0, buf1, sel_v,
             sem0, sem1):
        cid = lax.axis_index("c")
        sid = lax.axis_index("s")
        wid = sid * 2 + cid
        b = wid // 16
        r0 = (wid % 16) * RW

        pltpu.sync_copy(sel_hbm, sel_v)
        sel = sel_v[...]

        @plsc.parallel_loop(0, nb // L, unroll=8)
        def _(i):
            hist_v[pl.ds(i * L, L)] = jnp.zeros((L,), jnp.int32)

        ones = jnp.ones((L,), jnp.int32)

        def process(buf):
            for j in range(cr):
                @plsc.parallel_loop(0, C // L, unroll=8)
                def _(i):
                    bits = plsc.bitcast(buf[j, pl.ds(i * L, L)], jnp.int32)
                    if pass1:
                        idx = lax.shift_right_logical(
                            bits & jnp.int32(0x7FFFFFFF), 15)
                        plsc.addupdate_scatter(hist_v, [idx], ones)
                    else:
                        idx = bits & jnp.int32(0x7FFF)
                        msk = (bits & jnp.int32(0x7FFF8000)) == sel
                        plsc.addupdate_scatter(hist_v, [idx], ones, mask=msk)

        def start(buf, sem, chunk):
            pltpu.make_async_copy(
                data_hbm.at[b, pl.ds(r0 + chunk * cr, cr), :], buf,
                sem).start()

        def wait(buf, sem):
            pltpu.make_async_copy(
                data_hbm.at[b, pl.ds(r0, cr), :], buf, sem).wait()

        start(buf0, sem0, 0)

        def outer(g2, c):
            g = g2 * 2
            start(buf1, sem1, g + 1)
            wait(buf0, sem0)
            process(buf0)

            @pl.when(g + 2 < nch)
            def _():
                start(buf0, sem0, g + 2)

            wait(buf1, sem1)
            process(buf1)
            return c

        lax.fori_loop(0, nch // 2, outer, 0)

        pltpu.sync_copy(hist_v, out_hbm.at[wid])

    return pl.kernel(
        body,
        out_type=jax.ShapeDtypeStruct((NW, nb), jnp.int32),
        mesh=plsc.VectorSubcoreMesh(core_axis_name="c", subcore_axis_name="s"),
        compiler_params=pltpu.CompilerParams(needs_layout_passes=False),
        scratch_types=[
            pltpu.VMEM((nb,), jnp.int32),
            pltpu.VMEM((cr, C), jnp.float32),
            pltpu.VMEM((cr, C), jnp.float32),
            pltpu.VMEM((L,), jnp.int32),
            pltpu.SemaphoreType.DMA,
            pltpu.SemaphoreType.DMA,
        ],
    )


_hist1 = _make_hist(True)
_hist2 = _make_hist(False)


def _select_bin(hist, rank):
    c = jnp.cumsum(hist)
    bsel = jnp.argmax(c >= rank).astype(jnp.int32)
    below = jnp.where(bsel > 0, c[jnp.maximum(bsel - 1, 0)], 0)
    return bsel, below


def kernel(input):
    zero_sel = jnp.zeros((L,), jnp.int32)
    b1, below1 = _select_bin(_hist1(input, zero_sel).sum(axis=0), K)
    part2 = _hist2(input, zero_sel + (b1 << 15)).sum(axis=0)
    b2, _ = _select_bin(part2, K - below1)
    return lax.bitcast_convert_type((b1 << 15) | b2, jnp.float32)
